# baseline (speedup 1.0000x reference)
import functools

import jax
import jax.numpy as jnp
from jax import lax
from jax.experimental import pallas as pl
from jax.experimental.pallas import tpu as pltpu
from jax.experimental.pallas import tpu_sc as plsc

_B = 4096
_F = 26
_V = 100000
_D = 16
_C = 13
_H = 128
_EPS = 1e-5

_NC = 2
_NS = 16
_NW = _NC * _NS
_N = _B * _F
_BPW = _N // _NW

_HI = jax.lax.Precision.HIGHEST


def _gather_body(e2_hbm, e1_hbm, idx_hbm, o2_hbm, o1_hbm,
                 idx_v, e2_v, e1_v, sem2, sem1):
    wid = lax.axis_index("s") * _NC + lax.axis_index("c")
    base = wid * _BPW
    pltpu.sync_copy(idx_hbm.at[pl.ds(base, _BPW)], idx_v)
    c2 = pltpu.async_copy(e2_hbm.at[idx_v], e2_v, sem2)
    c1 = pltpu.async_copy(e1_hbm.at[idx_v], e1_v, sem1)
    c2.wait()
    pltpu.sync_copy(e2_v, o2_hbm.at[pl.ds(base, _BPW)])
    c1.wait()
    pltpu.sync_copy(e1_v, o1_hbm.at[pl.ds(base, _BPW)])


def _sc_gather(e2f, e1f, idx):
    mesh = plsc.VectorSubcoreMesh(core_axis_name="c", subcore_axis_name="s")
    k = pl.kernel(
        _gather_body,
        out_type=(
            jax.ShapeDtypeStruct((_N, _D), jnp.float32),
            jax.ShapeDtypeStruct((_N,), jnp.float32),
        ),
        mesh=mesh,
        scratch_types=[
            pltpu.VMEM((_BPW,), jnp.int32),
            pltpu.VMEM((_BPW, _D), jnp.float32),
            pltpu.VMEM((_BPW,), jnp.float32),
            pltpu.SemaphoreType.DMA,
            pltpu.SemaphoreType.DMA,
        ],
        compiler_params=pltpu.CompilerParams(use_tc_tiling_on_sc=False),
    )
    return k(e2f, e1f, idx)


def _bn(x, g, b):
    m = jnp.mean(x, axis=0, keepdims=True)
    xc = x - m
    v = jnp.mean(xc * xc, axis=0, keepdims=True)
    return g * xc / jnp.sqrt(v + _EPS) + b


def _dot(a, b):
    return jax.lax.dot(a, b, precision=_HI, preferred_element_type=jnp.float32)


def _tc_body(dnn0_ref, e1_ref, xi_ref, w1t_ref, b1_ref,
             wdt_ref, bd_ref, gd_ref, bed_ref,
             wat_ref, ba_ref, ga_ref, bea_ref,
             wbt_ref, bb_ref, gb_ref, beb_ref,
             wct_ref, bc_ref, out_ref):
    xi = xi_ref[...]
    dnn0 = dnn0_ref[...]

    fm1 = jnp.sum(e1_ref[...], axis=1, keepdims=True)
    fm1 = fm1 + _dot(xi, w1t_ref[...]) + b1_ref[...]

    rows = jax.lax.broadcasted_iota(jnp.int32, (_F * _D, _D), 0)
    cols = jax.lax.broadcasted_iota(jnp.int32, (_F * _D, _D), 1)
    sel = (rows % _D == cols).astype(jnp.float32)
    ssum = _dot(dnn0, sel)
    sqsum = _dot(dnn0 * dnn0, sel)
    fm2 = 0.5 * jnp.sum(ssum * ssum - sqsum, axis=1, keepdims=True)

    d = _dot(xi, wdt_ref[...]) + bd_ref[...]
    d = jax.nn.relu(_bn(d, gd_ref[...], bed_ref[...]))
    h = dnn0 + d
    h = _dot(h, wat_ref[...]) + ba_ref[...]
    h = jax.nn.relu(_bn(h, ga_ref[...], bea_ref[...]))
    h = _dot(h, wbt_ref[...]) + bb_ref[...]
    h = jax.nn.relu(_bn(h, gb_ref[...], beb_ref[...]))
    dnn_out = _dot(h, wct_ref[...]) + bc_ref[...]

    out_ref[...] = jax.nn.sigmoid(fm1 + fm2 + dnn_out)


def kernel(xi, xv, W1, b1, emb1, emb2, Wd, bd, gd, bed,
           Wa, ba, ga, bea, Wb, bb, gb, beb, Wc, bc):
    e2f = emb2.reshape(_F * _V, _D)
    e1f = emb1.reshape(_F * _V)
    idx = (xv + (jnp.arange(_F, dtype=jnp.int32) * _V)[None, :]).reshape(_N)

    e2g, e1g = _sc_gather(e2f, e1f, idx)

    dnn0 = e2g.reshape(_B, _F * _D)
    e1 = e1g.reshape(_B, _F)

    out = pl.pallas_call(
        _tc_body,
        out_shape=jax.ShapeDtypeStruct((_B, 1), jnp.float32),
    )(
        dnn0, e1, xi,
        W1.T, b1.reshape(1, 1),
        Wd.T, bd.reshape(1, -1), gd.reshape(1, -1), bed.reshape(1, -1),
        Wa.T, ba.reshape(1, -1), ga.reshape(1, -1), bea.reshape(1, -1),
        Wb.T, bb.reshape(1, -1), gb.reshape(1, -1), beb.reshape(1, -1),
        Wc.T, bc.reshape(1, 1),
    )
    return out
